# Initial kernel scaffold; baseline (speedup 1.0000x reference)
import jax
import jax.numpy as jnp
from jax.experimental import pallas as pl


def kernel(edge_attr, bond_embedding, stereo_embedding, conj_embedding):
    raise NotImplementedError("write your pallas kernel here")


# speedup vs baseline: 1.0862x; 1.0862x over previous
import functools

import jax
import jax.numpy as jnp
from jax import lax
from jax.experimental import pallas as pl
from jax.experimental.pallas import tpu as pltpu
from jax.experimental.pallas import tpu_sc as plsc

_E = 320000
_D = 128
_NB, _NS, _NC = 22, 6, 2
_NCOMBO = _NB * _NS * _NC

_NCORES = 2
_NSUB = 16
_NW = _NCORES * _NSUB
_EPW = _E // _NW
_L = 16
_CH = 80
_NCHUNK = _EPW // _CH


def _table_body(b_ref, s_ref, c_ref, t_ref):
    rid_b = lax.broadcasted_iota(jnp.int32, (_NCOMBO, _NB), 0)
    cid_b = lax.broadcasted_iota(jnp.int32, (_NCOMBO, _NB), 1)
    ohb = (rid_b // (_NS * _NC) == cid_b).astype(jnp.float32)
    rid_s = lax.broadcasted_iota(jnp.int32, (_NCOMBO, _NS), 0)
    cid_s = lax.broadcasted_iota(jnp.int32, (_NCOMBO, _NS), 1)
    ohs = ((rid_s // _NC) % _NS == cid_s).astype(jnp.float32)
    rid_c = lax.broadcasted_iota(jnp.int32, (_NCOMBO, _NC), 0)
    cid_c = lax.broadcasted_iota(jnp.int32, (_NCOMBO, _NC), 1)
    ohc = (rid_c % _NC == cid_c).astype(jnp.float32)
    t_ref[...] = (
        jnp.dot(ohb, b_ref[...], preferred_element_type=jnp.float32)
        + jnp.dot(ohs, s_ref[...], preferred_element_type=jnp.float32)
        + jnp.dot(ohc, c_ref[...], preferred_element_type=jnp.float32)
    )


def _build_table(bond, stereo, conj):
    return pl.pallas_call(
        _table_body,
        out_shape=jax.ShapeDtypeStruct((_NCOMBO, _D), jnp.float32),
    )(bond, stereo, conj)


def _sc_body(t_hbm, ea0_hbm, ea1_hbm, ea2_hbm, out_hbm,
             ea0_v, ea1_v, ea2_v, code_v, rows_v, gat_sem):
    wid = lax.axis_index("s") * _NCORES + lax.axis_index("c")
    base = wid * _EPW

    pltpu.sync_copy(ea0_hbm.at[pl.ds(base, _EPW)], ea0_v)
    pltpu.sync_copy(ea1_hbm.at[pl.ds(base, _EPW)], ea1_v)
    pltpu.sync_copy(ea2_hbm.at[pl.ds(base, _EPW)], ea2_v)

    def code_step(i, carry):
        sl = pl.ds(i * _L, _L)
        code_v[sl] = ea0_v[sl] * (_NS * _NC) + ea1_v[sl] * _NC + ea2_v[sl]
        return carry

    lax.fori_loop(0, _EPW // _L, code_step, 0)

    def chunk_step(g, carry):
        cb = g * _CH
        pltpu.async_copy(
            t_hbm.at[code_v.at[pl.ds(cb, _CH)]], rows_v, gat_sem
        ).wait()
        pltpu.sync_copy(rows_v, out_hbm.at[pl.ds(base + cb, _CH)])
        return carry

    lax.fori_loop(0, _NCHUNK, chunk_step, 0)


_sc_gather = functools.partial(
    pl.kernel,
    out_type=jax.ShapeDtypeStruct((_E, _D), jnp.float32),
    mesh=plsc.VectorSubcoreMesh(core_axis_name="c", subcore_axis_name="s"),
    scratch_types=[
        pltpu.VMEM((_EPW,), jnp.int32),
        pltpu.VMEM((_EPW,), jnp.int32),
        pltpu.VMEM((_EPW,), jnp.int32),
        pltpu.VMEM((_EPW,), jnp.int32),
        pltpu.VMEM((_CH, _D), jnp.float32),
        pltpu.SemaphoreType.DMA,
    ],
)(_sc_body)


@jax.jit
def kernel(edge_attr, bond_embedding, stereo_embedding, conj_embedding):
    t = _build_table(bond_embedding, stereo_embedding, conj_embedding)
    ea0 = edge_attr[:, 0].astype(jnp.int32)
    ea1 = edge_attr[:, 1].astype(jnp.int32)
    ea2 = edge_attr[:, 2].astype(jnp.int32)
    return _sc_gather(t, ea0, ea1, ea2)


# speedup vs baseline: 1.0952x; 1.0083x over previous
import functools

import jax
import jax.numpy as jnp
from jax import lax
from jax.experimental import pallas as pl
from jax.experimental.pallas import tpu as pltpu
from jax.experimental.pallas import tpu_sc as plsc

_E = 320000
_D = 128
_NB, _NS, _NC = 22, 6, 2
_NCOMBO = _NB * _NS * _NC

_NCORES = 2
_NSUB = 16
_NW = _NCORES * _NSUB
_EPW = _E // _NW
_L = 16
_CH = 80
_NCHUNK = _EPW // _CH
_NBUF = 6
_LAG = 2


def _table_body(b_ref, s_ref, c_ref, t_ref):
    rid_b = lax.broadcasted_iota(jnp.int32, (_NCOMBO, _NB), 0)
    cid_b = lax.broadcasted_iota(jnp.int32, (_NCOMBO, _NB), 1)
    ohb = (rid_b // (_NS * _NC) == cid_b).astype(jnp.float32)
    rid_s = lax.broadcasted_iota(jnp.int32, (_NCOMBO, _NS), 0)
    cid_s = lax.broadcasted_iota(jnp.int32, (_NCOMBO, _NS), 1)
    ohs = ((rid_s // _NC) % _NS == cid_s).astype(jnp.float32)
    rid_c = lax.broadcasted_iota(jnp.int32, (_NCOMBO, _NC), 0)
    cid_c = lax.broadcasted_iota(jnp.int32, (_NCOMBO, _NC), 1)
    ohc = (rid_c % _NC == cid_c).astype(jnp.float32)
    t_ref[...] = (
        jnp.dot(ohb, b_ref[...], preferred_element_type=jnp.float32)
        + jnp.dot(ohs, s_ref[...], preferred_element_type=jnp.float32)
        + jnp.dot(ohc, c_ref[...], preferred_element_type=jnp.float32)
    )


def _build_table(bond, stereo, conj):
    return pl.pallas_call(
        _table_body,
        out_shape=jax.ShapeDtypeStruct((_NCOMBO, _D), jnp.float32),
    )(bond, stereo, conj)


def _sc_body(t_hbm, ea0_hbm, ea1_hbm, ea2_hbm, out_hbm,
             ea0_v, ea1_v, ea2_v, code_v, rows_v, gat_sem, wr_sem):
    wid = lax.axis_index("s") * _NCORES + lax.axis_index("c")
    base = wid * _EPW

    pltpu.sync_copy(ea0_hbm.at[pl.ds(base, _EPW)], ea0_v)
    pltpu.sync_copy(ea1_hbm.at[pl.ds(base, _EPW)], ea1_v)
    pltpu.sync_copy(ea2_hbm.at[pl.ds(base, _EPW)], ea2_v)

    def code_step(i, carry):
        sl = pl.ds(i * _L, _L)
        code_v[sl] = ea0_v[sl] * (_NS * _NC) + ea1_v[sl] * _NC + ea2_v[sl]
        return carry

    lax.fori_loop(0, _EPW // _L, code_step, 0)

    def g_desc(g, b):
        return pltpu.make_async_copy(
            t_hbm.at[code_v.at[pl.ds(g * _CH, _CH)]], rows_v.at[b],
            gat_sem.at[b])

    def w_desc(g, b):
        return pltpu.make_async_copy(
            rows_v.at[b], out_hbm.at[pl.ds(base + g * _CH, _CH)],
            wr_sem.at[b])

    for g in range(_NBUF):
        g_desc(g, g).start()

    def chunk_step(g, carry):
        b = lax.rem(g, _NBUF)
        g_desc(g, b).wait()
        w_desc(g, b).start()
        j = g - _LAG

        def reclaim(j):
            bj = lax.rem(j, _NBUF)
            w_desc(j, bj).wait()
            g_desc(j + _NBUF, bj).start()
            return 0

        lax.cond((j >= 0) & (j + _NBUF < _NCHUNK), reclaim, lambda j: 0, j)
        return carry

    lax.fori_loop(0, _NCHUNK, chunk_step, 0)

    for j in range(_NCHUNK - _NBUF, _NCHUNK):
        w_desc(j, j % _NBUF).wait()


_sc_gather = functools.partial(
    pl.kernel,
    out_type=jax.ShapeDtypeStruct((_E, _D), jnp.float32),
    mesh=plsc.VectorSubcoreMesh(core_axis_name="c", subcore_axis_name="s"),
    scratch_types=[
        pltpu.VMEM((_EPW,), jnp.int32),
        pltpu.VMEM((_EPW,), jnp.int32),
        pltpu.VMEM((_EPW,), jnp.int32),
        pltpu.VMEM((_EPW,), jnp.int32),
        pltpu.VMEM((_NBUF, _CH, _D), jnp.float32),
        pltpu.SemaphoreType.DMA((_NBUF,)),
        pltpu.SemaphoreType.DMA((_NBUF,)),
    ],
)(_sc_body)


@jax.jit
def kernel(edge_attr, bond_embedding, stereo_embedding, conj_embedding):
    t = _build_table(bond_embedding, stereo_embedding, conj_embedding)
    ea0 = edge_attr[:, 0].astype(jnp.int32)
    ea1 = edge_attr[:, 1].astype(jnp.int32)
    ea2 = edge_attr[:, 2].astype(jnp.int32)
    return _sc_gather(t, ea0, ea1, ea2)


# speedup vs baseline: 6.1962x; 5.6575x over previous
import functools

import jax
import jax.numpy as jnp
from jax import lax
from jax.experimental import pallas as pl
from jax.experimental.pallas import tpu as pltpu
from jax.experimental.pallas import tpu_sc as plsc

_E = 320000
_D = 128
_NB, _NS, _NC = 22, 6, 2
_NCOMBO = _NB * _NS * _NC

_NCORES = 2
_NSUB = 16
_NW = _NCORES * _NSUB
_EPW = _E // _NW
_L = 16
_CH = 80
_NCHUNK = _EPW // _CH
_NBUF = 4
_LAG = 2


def _table_body(b_ref, s_ref, c_ref, t_ref):
    rid_b = lax.broadcasted_iota(jnp.int32, (_NCOMBO, _NB), 0)
    cid_b = lax.broadcasted_iota(jnp.int32, (_NCOMBO, _NB), 1)
    ohb = (rid_b // (_NS * _NC) == cid_b).astype(jnp.float32)
    rid_s = lax.broadcasted_iota(jnp.int32, (_NCOMBO, _NS), 0)
    cid_s = lax.broadcasted_iota(jnp.int32, (_NCOMBO, _NS), 1)
    ohs = ((rid_s // _NC) % _NS == cid_s).astype(jnp.float32)
    rid_c = lax.broadcasted_iota(jnp.int32, (_NCOMBO, _NC), 0)
    cid_c = lax.broadcasted_iota(jnp.int32, (_NCOMBO, _NC), 1)
    ohc = (rid_c % _NC == cid_c).astype(jnp.float32)
    t_ref[...] = (
        jnp.dot(ohb, b_ref[...], preferred_element_type=jnp.float32)
        + jnp.dot(ohs, s_ref[...], preferred_element_type=jnp.float32)
        + jnp.dot(ohc, c_ref[...], preferred_element_type=jnp.float32)
    )


def _build_table(bond, stereo, conj):
    return pl.pallas_call(
        _table_body,
        out_shape=jax.ShapeDtypeStruct((_NCOMBO, _D), jnp.float32),
    )(bond, stereo, conj)


def _sc_body(t_hbm, ea0_hbm, ea1_hbm, ea2_hbm, out_hbm,
             t_v, ea0_v, ea1_v, ea2_v, code_v, rows_v, wr_sem):
    wid = lax.axis_index("s") * _NCORES + lax.axis_index("c")
    base = wid * _EPW

    pltpu.sync_copy(t_hbm, t_v)
    pltpu.sync_copy(ea0_hbm.at[pl.ds(base, _EPW)], ea0_v)
    pltpu.sync_copy(ea1_hbm.at[pl.ds(base, _EPW)], ea1_v)
    pltpu.sync_copy(ea2_hbm.at[pl.ds(base, _EPW)], ea2_v)

    def code_step(i, carry):
        sl = pl.ds(i * _L, _L)
        code_v[sl] = (ea0_v[sl] * (_NS * _NC) + ea1_v[sl] * _NC
                      + ea2_v[sl]) * _D
        return carry

    lax.fori_loop(0, _EPW // _L, code_step, 0)

    def w_desc(g, b):
        return pltpu.make_async_copy(
            rows_v.at[pl.ds(b * _CH, _CH)],
            out_hbm.at[pl.ds(base + g * _CH, _CH)],
            wr_sem.at[b])

    col_off = [jnp.arange(_L, dtype=jnp.int32) + d * _L for d in range(8)]
    lane_i = [jnp.full((_L, 1), i, dtype=jnp.int32) for i in range(_L)]
    dnums = lax.GatherDimensionNumbers(
        offset_dims=(), collapsed_slice_dims=(0,), start_index_map=(0,))

    def splat(vec, i):
        return lax.gather(vec, lane_i[i], dnums, (1,),
                          mode=lax.GatherScatterMode.PROMISE_IN_BOUNDS)

    def chunk_step(g, carry):
        b = lax.rem(g, _NBUF)

        def reclaim(j):
            w_desc(j, lax.rem(j, _NBUF)).wait()
            return 0

        lax.cond(g >= _NBUF, reclaim, lambda j: 0, g - _NBUF)

        def fill_group(e16, carry):
            codes = code_v[pl.ds(g * _CH + e16 * _L, _L)]
            for i in range(_L):
                rb = splat(codes, i)
                e = b * _CH + e16 * _L + i
                for d in range(8):
                    seg = plsc.load_gather(t_v, [rb + col_off[d]])
                    rows_v[e, pl.ds(d * _L, _L)] = seg
            return carry

        lax.fori_loop(0, _CH // _L, fill_group, 0)
        w_desc(g, b).start()
        return carry

    lax.fori_loop(0, _NCHUNK, chunk_step, 0)

    for j in range(_NCHUNK - _NBUF, _NCHUNK):
        w_desc(j, j % _NBUF).wait()


_sc_gather = functools.partial(
    pl.kernel,
    out_type=jax.ShapeDtypeStruct((_E, _D), jnp.float32),
    mesh=plsc.VectorSubcoreMesh(core_axis_name="c", subcore_axis_name="s"),
    compiler_params=pltpu.CompilerParams(needs_layout_passes=False),
    scratch_types=[
        pltpu.VMEM((_NCOMBO * _D,), jnp.float32),
        pltpu.VMEM((_EPW,), jnp.int32),
        pltpu.VMEM((_EPW,), jnp.int32),
        pltpu.VMEM((_EPW,), jnp.int32),
        pltpu.VMEM((_EPW,), jnp.int32),
        pltpu.VMEM((_NBUF * _CH, _D), jnp.float32),
        pltpu.SemaphoreType.DMA((_NBUF,)),
    ],
)(_sc_body)


@jax.jit
def kernel(edge_attr, bond_embedding, stereo_embedding, conj_embedding):
    t = _build_table(bond_embedding, stereo_embedding, conj_embedding)
    ea0 = edge_attr[:, 0].astype(jnp.int32)
    ea1 = edge_attr[:, 1].astype(jnp.int32)
    ea2 = edge_attr[:, 2].astype(jnp.int32)
    return _sc_gather(t.reshape(-1), ea0, ea1, ea2)


# speedup vs baseline: 9.0482x; 1.4603x over previous
import functools

import jax
import jax.numpy as jnp
from jax import lax
from jax.experimental import pallas as pl
from jax.experimental.pallas import tpu as pltpu
from jax.experimental.pallas import tpu_sc as plsc

_E = 320000
_D = 128
_NB, _NS, _NC = 22, 6, 2
_NCOMBO = _NB * _NS * _NC

_NCORES = 2
_NSUB = 16
_NW = _NCORES * _NSUB
_EPW = _E // _NW
_L = 16
_CH = 80
_NCHUNK = _EPW // _CH
_NBUF = 4
_LAG = 2


def _table_body(b_ref, s_ref, c_ref, t_ref):
    rid_b = lax.broadcasted_iota(jnp.int32, (_NCOMBO, _NB), 0)
    cid_b = lax.broadcasted_iota(jnp.int32, (_NCOMBO, _NB), 1)
    ohb = (rid_b // (_NS * _NC) == cid_b).astype(jnp.float32)
    rid_s = lax.broadcasted_iota(jnp.int32, (_NCOMBO, _NS), 0)
    cid_s = lax.broadcasted_iota(jnp.int32, (_NCOMBO, _NS), 1)
    ohs = ((rid_s // _NC) % _NS == cid_s).astype(jnp.float32)
    rid_c = lax.broadcasted_iota(jnp.int32, (_NCOMBO, _NC), 0)
    cid_c = lax.broadcasted_iota(jnp.int32, (_NCOMBO, _NC), 1)
    ohc = (rid_c % _NC == cid_c).astype(jnp.float32)
    t_ref[...] = (
        jnp.dot(ohb, b_ref[...], preferred_element_type=jnp.float32)
        + jnp.dot(ohs, s_ref[...], preferred_element_type=jnp.float32)
        + jnp.dot(ohc, c_ref[...], preferred_element_type=jnp.float32)
    )


def _build_table(bond, stereo, conj):
    return pl.pallas_call(
        _table_body,
        out_shape=jax.ShapeDtypeStruct((_NCOMBO, _D), jnp.float32),
    )(bond, stereo, conj)


def _sc_body(t_hbm, ea0_hbm, ea1_hbm, ea2_hbm, out_hbm,
             t_v, ea0_v, ea1_v, ea2_v, code_v, rows_v, wr_sem):
    wid = lax.axis_index("s") * _NCORES + lax.axis_index("c")
    base = wid * _EPW

    pltpu.sync_copy(t_hbm, t_v)
    pltpu.sync_copy(ea0_hbm.at[pl.ds(base, _EPW)], ea0_v)
    pltpu.sync_copy(ea1_hbm.at[pl.ds(base, _EPW)], ea1_v)
    pltpu.sync_copy(ea2_hbm.at[pl.ds(base, _EPW)], ea2_v)

    def code_step(i, carry):
        sl = pl.ds(i * _L, _L)
        code_v[sl] = (ea0_v[sl] * (_NS * _NC) + ea1_v[sl] * _NC
                      + ea2_v[sl]) * _D
        return carry

    lax.fori_loop(0, _EPW // _L, code_step, 0)

    def w_desc(g, b):
        return pltpu.make_async_copy(
            rows_v.at[pl.ds(b * _CH, _CH)],
            out_hbm.at[pl.ds(base + g * _CH, _CH)],
            wr_sem.at[b])

    col_off = [jnp.arange(_L, dtype=jnp.int32) + d * _L for d in range(8)]
    lane_i = [jnp.full((_L, 1), i, dtype=jnp.int32) for i in range(_L)]
    dnums = lax.GatherDimensionNumbers(
        offset_dims=(), collapsed_slice_dims=(0,), start_index_map=(0,))

    def splat(vec, i):
        return lax.gather(vec, lane_i[i], dnums, (1,),
                          mode=lax.GatherScatterMode.PROMISE_IN_BOUNDS)

    def chunk_step(g, carry):
        b = lax.rem(g, _NBUF)

        def reclaim(j):
            w_desc(j, lax.rem(j, _NBUF)).wait()
            return 0

        lax.cond(g >= _NBUF, reclaim, lambda j: 0, g - _NBUF)

        @plsc.parallel_loop(0, _CH // _L, unroll=2)
        def fill_group(e16):
            codes = code_v[pl.ds(g * _CH + e16 * _L, _L)]
            bases = [splat(codes, i) for i in range(_L)]
            for d in range(8):
                for i in range(_L):
                    seg = plsc.load_gather(t_v, [bases[i] + col_off[d]])
                    rows_v[b * _CH + e16 * _L + i, pl.ds(d * _L, _L)] = seg
        w_desc(g, b).start()
        return carry

    lax.fori_loop(0, _NCHUNK, chunk_step, 0)

    for j in range(_NCHUNK - _NBUF, _NCHUNK):
        w_desc(j, j % _NBUF).wait()


_sc_gather = functools.partial(
    pl.kernel,
    out_type=jax.ShapeDtypeStruct((_E, _D), jnp.float32),
    mesh=plsc.VectorSubcoreMesh(core_axis_name="c", subcore_axis_name="s"),
    compiler_params=pltpu.CompilerParams(needs_layout_passes=False),
    scratch_types=[
        pltpu.VMEM((_NCOMBO * _D,), jnp.float32),
        pltpu.VMEM((_EPW,), jnp.int32),
        pltpu.VMEM((_EPW,), jnp.int32),
        pltpu.VMEM((_EPW,), jnp.int32),
        pltpu.VMEM((_EPW,), jnp.int32),
        pltpu.VMEM((_NBUF * _CH, _D), jnp.float32),
        pltpu.SemaphoreType.DMA((_NBUF,)),
    ],
)(_sc_body)


@jax.jit
def kernel(edge_attr, bond_embedding, stereo_embedding, conj_embedding):
    t = _build_table(bond_embedding, stereo_embedding, conj_embedding)
    ea0 = edge_attr[:, 0].astype(jnp.int32)
    ea1 = edge_attr[:, 1].astype(jnp.int32)
    ea2 = edge_attr[:, 2].astype(jnp.int32)
    return _sc_gather(t.reshape(-1), ea0, ea1, ea2)
